# baseline (speedup 1.0000x reference)
import functools

import jax
import jax.numpy as jnp
from jax import lax
from jax.experimental import pallas as pl
from jax.experimental.pallas import tpu as pltpu
from jax.experimental.pallas import tpu_sc as plsc

_N = 10000
_E = 320000
_NSC = 2
_NSUB = 16
_W = _NSC * _NSUB
_EPW = _E // _W
_K = 40
_NCHUNK = _EPW // _K
_RB = 624
_ZCOPIES = 16


def _make_edge_kernel(D):
    mesh = plsc.VectorSubcoreMesh(core_axis_name="c", subcore_axis_name="s")
    groups = D // 16

    @functools.partial(
        pl.kernel,
        out_type=jax.ShapeDtypeStruct((_NSC, _N, D), jnp.float32),
        mesh=mesh,
        compiler_params=pltpu.CompilerParams(use_tc_tiling_on_sc=(D == 128)),
        scratch_types=[
            pltpu.VMEM((D,), jnp.float32),
            pltpu.VMEM((4, _K), jnp.int32),
            pltpu.VMEM((4, _K), jnp.int32),
            pltpu.VMEM((2, _K, D), jnp.float32),
            pltpu.VMEM((2, _K, D), jnp.float32),
            pltpu.VMEM((2, _K, D), jnp.float32),
            pltpu.VMEM_SHARED((_N, D), jnp.float32),
            [pltpu.SemaphoreType.DMA] * 6,
            [pltpu.SemaphoreType.DMA] * 4,
        ],
    )
    def edge_kernel(a_hbm, b_hbm, c_hbm, src_hbm, dst_hbm, bm_hbm, out_hbm,
                    bm_v, src_v, dst_v, bufa, bufb, bufc, agg_sh,
                    gsems, isems):
        cc = lax.axis_index("c")
        s = lax.axis_index("s")
        w = cc * _NSUB + s
        base = w * _EPW
        pltpu.sync_copy(bm_hbm, bm_v)
        bmv = [bm_v[pl.ds(16 * g, 16)] for g in range(groups)]

        def zero_body(i, carry):
            for g in range(groups):
                bufa[0, i, pl.ds(g * 16, 16)] = jnp.zeros((16,), jnp.float32)
            return carry
        lax.fori_loop(0, _K, zero_body, 0)
        for r in range(_ZCOPIES):
            pltpu.sync_copy(bufa.at[0],
                            agg_sh.at[pl.ds(s * _RB + r * _K, _K)])
        plsc.subcore_barrier()

        def load_idx(k, ci):
            off = base + ci * _K
            pltpu.async_copy(src_hbm.at[pl.ds(off, _K)], src_v.at[k],
                             isems[k])
            pltpu.async_copy(dst_hbm.at[pl.ds(off, _K)], dst_v.at[k],
                             isems[k])

        def wait_idx(k):
            pltpu.make_async_copy(src_hbm.at[pl.ds(0, _K)], src_v.at[k],
                                  isems[k]).wait()
            pltpu.make_async_copy(src_hbm.at[pl.ds(0, _K)], dst_v.at[k],
                                  isems[k]).wait()

        def issue(p, k, ci):
            pltpu.async_copy(a_hbm.at[src_v.at[k]], bufa.at[p], gsems[3 * p])
            pltpu.async_copy(b_hbm.at[dst_v.at[k]], bufb.at[p],
                             gsems[3 * p + 1])
            pltpu.async_copy(c_hbm.at[pl.ds(base + ci * _K, _K)], bufc.at[p],
                             gsems[3 * p + 2])

        def process(p, k):
            pltpu.make_async_copy(a_hbm.at[src_v.at[k]], bufa.at[p],
                                  gsems[3 * p]).wait()
            pltpu.make_async_copy(b_hbm.at[dst_v.at[k]], bufb.at[p],
                                  gsems[3 * p + 1]).wait()
            pltpu.make_async_copy(c_hbm.at[pl.ds(0, _K)], bufc.at[p],
                                  gsems[3 * p + 2]).wait()

            def edge_body(e, carry2):
                for g in range(groups):
                    sl = pl.ds(g * 16, 16)
                    v = (bufa[p, e, sl] + bufb[p, e, sl]
                         + bufc[p, e, sl]) + bmv[g]
                    bufa[p, e, sl] = jnp.maximum(v, jnp.float32(0.0))
                return carry2
            lax.fori_loop(0, _K, edge_body, 0)
            pltpu.sync_copy(bufa.at[p], agg_sh.at[dst_v.at[k]], add=True)

        load_idx(0, 0)
        load_idx(1, 1)
        wait_idx(0)
        issue(0, 0, 0)
        wait_idx(1)
        issue(1, 1, 1)

        def body(i, carry):
            c0 = 4 * i
            load_idx(2, c0 + 2)
            load_idx(3, c0 + 3)
            process(0, 0)
            wait_idx(2)
            issue(0, 2, c0 + 2)
            load_idx(0, c0 + 4)
            process(1, 1)
            wait_idx(3)
            issue(1, 3, c0 + 3)
            load_idx(1, c0 + 5)
            process(0, 2)
            wait_idx(0)
            issue(0, 0, c0 + 4)
            process(1, 3)
            wait_idx(1)
            issue(1, 1, c0 + 5)
            return carry
        lax.fori_loop(0, (_NCHUNK - 2) // 4, body, 0)
        process(0, 0)
        process(1, 1)

        plsc.subcore_barrier()
        for r in range(_ZCOPIES):
            row0 = s * _RB + r * _K
            pltpu.sync_copy(agg_sh.at[pl.ds(row0, _K)],
                            out_hbm.at[cc, pl.ds(row0, _K)])

    return edge_kernel


_edge128 = _make_edge_kernel(128)
_edge16 = _make_edge_kernel(16)


def _ab_call(h, ws, wd, D):
    R = 2000

    def body(x_ref, ws_ref, wd_ref, a_ref, b_ref):
        xv = x_ref[...]
        a_ref[...] = jnp.dot(xv, ws_ref[...],
                             preferred_element_type=jnp.float32)
        b_ref[...] = jnp.dot(xv, wd_ref[...],
                             preferred_element_type=jnp.float32)

    return pl.pallas_call(
        body,
        grid=(_N // R,),
        in_specs=[
            pl.BlockSpec((R, 128), lambda i: (i, 0)),
            pl.BlockSpec((128, D), lambda i: (0, 0)),
            pl.BlockSpec((128, D), lambda i: (0, 0)),
        ],
        out_specs=[pl.BlockSpec((R, D), lambda i: (i, 0)),
                   pl.BlockSpec((R, D), lambda i: (i, 0))],
        out_shape=[jax.ShapeDtypeStruct((_N, D), jnp.float32)] * 2,
    )(h, ws, wd)


def _c_call(ea, we, D):
    RB = 4000

    def body(ea_ref, we_ref, c_ref):
        c_ref[...] = jnp.dot(ea_ref[...], we_ref[...],
                             preferred_element_type=jnp.float32)

    return pl.pallas_call(
        body,
        grid=(_E // RB,),
        in_specs=[pl.BlockSpec((RB, 16), lambda i: (i, 0)),
                  pl.BlockSpec((16, D), lambda i: (0, 0))],
        out_specs=pl.BlockSpec((RB, D), lambda i: (i, 0)),
        out_shape=jax.ShapeDtypeStruct((_E, D), jnp.float32),
    )(ea, we)


def _node_call(h, a0, a1, wn, bn, act, D):
    R = 2000

    def body(x_ref, a0_ref, a1_ref, wn_ref, bn_ref, o_ref):
        agg = a0_ref[...] + a1_ref[...]
        xcat = jnp.concatenate([x_ref[...], agg], axis=1)
        out = jnp.dot(xcat, wn_ref[...],
                      preferred_element_type=jnp.float32) + bn_ref[...]
        if act == "relu":
            out = jnp.maximum(out, jnp.float32(0.0))
        elif act == "sigmoid":
            out = jax.nn.sigmoid(out)
        o_ref[...] = out

    return pl.pallas_call(
        body,
        grid=(_N // R,),
        in_specs=[
            pl.BlockSpec((R, 128), lambda i: (i, 0)),
            pl.BlockSpec((R, D), lambda i: (i, 0)),
            pl.BlockSpec((R, D), lambda i: (i, 0)),
            pl.BlockSpec((128 + D, D), lambda i: (0, 0)),
            pl.BlockSpec((1, D), lambda i: (0, 0)),
        ],
        out_specs=pl.BlockSpec((R, D), lambda i: (i, 0)),
        out_shape=jax.ShapeDtypeStruct((_N, D), jnp.float32),
    )(h, a0, a1, wn, bn.reshape(1, D))


def kernel(x, edge_index, edge_attr, Wm1, bm1, Wn1, bn1, Wm2, bm2, Wn2, bn2,
           Wm3, bm3, Wn3, bn3, Wm4, bm4, Wn4, bn4):
    src = edge_index[0]
    dst = edge_index[1]

    h = x
    ea4, src4, dst4 = edge_attr, src, dst
    for i, (Wm, bm, Wn, bn) in enumerate(((Wm1, bm1, Wn1, bn1),
                                          (Wm2, bm2, Wn2, bn2),
                                          (Wm3, bm3, Wn3, bn3))):
        ws, wd, we = Wm[:128], Wm[128:256], Wm[256:272]
        a, b = _ab_call(h, ws, wd, 128)
        cc = _c_call(edge_attr, we, 128)
        aggp = _edge128(a, b, cc, src, dst, bm)
        h = _node_call(h, aggp[0], aggp[1], Wn, bn, "relu", 128)
        if i == 0:
            ea4, src4, dst4, _ = lax.optimization_barrier(
                (edge_attr, src, dst, h[0, 0]))

    ws4 = jnp.pad(Wm4[:128], ((0, 0), (0, 15)))
    wd4 = jnp.pad(Wm4[128:256], ((0, 0), (0, 15)))
    we4 = jnp.pad(Wm4[256:272], ((0, 0), (0, 15)))
    bm4p = jnp.pad(bm4, (0, 15))
    wn4 = jnp.pad(Wn4, ((0, 15), (0, 15)))
    bn4p = jnp.pad(bn4, (0, 15))

    a, b = _ab_call(h, ws4, wd4, 16)
    cc = _c_call(ea4, we4, 16)
    aggp = _edge16(a, b, cc, src4, dst4, bm4p)
    out16 = _node_call(h, aggp[0], aggp[1], wn4, bn4p, "sigmoid", 16)
    return out16[:, 0:1]


# speedup vs baseline: 1.0714x; 1.0714x over previous
import functools

import jax
import jax.numpy as jnp
from jax import lax
from jax.experimental import pallas as pl
from jax.experimental.pallas import tpu as pltpu
from jax.experimental.pallas import tpu_sc as plsc

_N = 10000
_E = 320000
_NSC = 2
_NSUB = 16
_W = _NSC * _NSUB
_EPW = _E // _W
_K = 40
_NCHUNK = _EPW // _K
_RB = 624
_ZCOPIES = 16


def _make_edge_kernel(D):
    mesh = plsc.VectorSubcoreMesh(core_axis_name="c", subcore_axis_name="s")
    groups = D // 16

    @functools.partial(
        pl.kernel,
        out_type=jax.ShapeDtypeStruct((_NSC, _N, D), jnp.float32),
        mesh=mesh,
        compiler_params=pltpu.CompilerParams(use_tc_tiling_on_sc=(D == 128)),
        scratch_types=[
            pltpu.VMEM((D,), jnp.float32),
            pltpu.VMEM((4, _K), jnp.int32),
            pltpu.VMEM((4, _K), jnp.int32),
            pltpu.VMEM((2, _K, D), jnp.float32),
            pltpu.VMEM((2, _K, D), jnp.float32),
            pltpu.VMEM((2, _K, D), jnp.float32),
            pltpu.VMEM_SHARED((_N, D), jnp.float32),
            [pltpu.SemaphoreType.DMA] * 6,
            [pltpu.SemaphoreType.DMA] * 4,
        ],
    )
    def edge_kernel(a_hbm, b_hbm, c_hbm, src_hbm, dst_hbm, bm_hbm, out_hbm,
                    bm_v, src_v, dst_v, bufa, bufb, bufc, agg_sh,
                    gsems, isems):
        cc = lax.axis_index("c")
        s = lax.axis_index("s")
        w = cc * _NSUB + s
        base = w * _EPW
        pltpu.sync_copy(bm_hbm, bm_v)
        bmv = [bm_v[pl.ds(16 * g, 16)] for g in range(groups)]

        def zero_body(i, carry):
            for g in range(groups):
                bufa[0, i, pl.ds(g * 16, 16)] = jnp.zeros((16,), jnp.float32)
            return carry
        lax.fori_loop(0, _K, zero_body, 0)
        for r in range(_ZCOPIES):
            pltpu.sync_copy(bufa.at[0],
                            agg_sh.at[pl.ds(s * _RB + r * _K, _K)])
        plsc.subcore_barrier()

        def load_idx(k, ci):
            off = base + ci * _K
            pltpu.async_copy(src_hbm.at[pl.ds(off, _K)], src_v.at[k],
                             isems[k])
            pltpu.async_copy(dst_hbm.at[pl.ds(off, _K)], dst_v.at[k],
                             isems[k])

        def wait_idx(k):
            pltpu.make_async_copy(src_hbm.at[pl.ds(0, _K)], src_v.at[k],
                                  isems[k]).wait()
            pltpu.make_async_copy(src_hbm.at[pl.ds(0, _K)], dst_v.at[k],
                                  isems[k]).wait()

        def issue(p, k, ci):
            pltpu.async_copy(a_hbm.at[src_v.at[k]], bufa.at[p], gsems[3 * p])
            pltpu.async_copy(b_hbm.at[dst_v.at[k]], bufb.at[p],
                             gsems[3 * p + 1])
            pltpu.async_copy(c_hbm.at[pl.ds(base + ci * _K, _K)], bufc.at[p],
                             gsems[3 * p + 2])

        def process(p, k):
            pltpu.make_async_copy(a_hbm.at[src_v.at[k]], bufa.at[p],
                                  gsems[3 * p]).wait()
            pltpu.make_async_copy(b_hbm.at[dst_v.at[k]], bufb.at[p],
                                  gsems[3 * p + 1]).wait()
            pltpu.make_async_copy(c_hbm.at[pl.ds(0, _K)], bufc.at[p],
                                  gsems[3 * p + 2]).wait()

            def edge_body(e, carry2):
                for g in range(groups):
                    sl = pl.ds(g * 16, 16)
                    v = (bufa[p, e, sl] + bufb[p, e, sl]
                         + bufc[p, e, sl]) + bmv[g]
                    bufa[p, e, sl] = jnp.maximum(v, jnp.float32(0.0))
                return carry2
            lax.fori_loop(0, _K, edge_body, 0)
            pltpu.sync_copy(bufa.at[p], agg_sh.at[dst_v.at[k]], add=True)

        load_idx(0, 0)
        load_idx(1, 1)
        wait_idx(0)
        issue(0, 0, 0)
        wait_idx(1)
        issue(1, 1, 1)

        def body(i, carry):
            c0 = 4 * i
            load_idx(2, c0 + 2)
            load_idx(3, c0 + 3)
            process(0, 0)
            wait_idx(2)
            issue(0, 2, c0 + 2)
            load_idx(0, c0 + 4)
            process(1, 1)
            wait_idx(3)
            issue(1, 3, c0 + 3)
            load_idx(1, c0 + 5)
            process(0, 2)
            wait_idx(0)
            issue(0, 0, c0 + 4)
            process(1, 3)
            wait_idx(1)
            issue(1, 1, c0 + 5)
            return carry
        lax.fori_loop(0, (_NCHUNK - 2) // 4, body, 0)
        process(0, 0)
        process(1, 1)

        plsc.subcore_barrier()
        for r in range(_ZCOPIES):
            row0 = s * _RB + r * _K
            pltpu.sync_copy(agg_sh.at[pl.ds(row0, _K)],
                            out_hbm.at[cc, pl.ds(row0, _K)])

    return edge_kernel


_edge128 = _make_edge_kernel(128)


_K1 = 80
_NCHUNK1 = _EPW // _K1
_NPAD1 = 10240

_mesh1 = plsc.VectorSubcoreMesh(core_axis_name="c", subcore_axis_name="s")


@functools.partial(
    pl.kernel,
    out_type=jax.ShapeDtypeStruct((_NSC, _NPAD1), jnp.float32),
    mesh=_mesh1,
    compiler_params=pltpu.CompilerParams(use_tc_tiling_on_sc=True,
                                         needs_layout_passes=False),
    scratch_types=[
        pltpu.VMEM((16,), jnp.float32),
        pltpu.VMEM((_N,), jnp.float32),
        pltpu.VMEM((_N,), jnp.float32),
        pltpu.VMEM((640,), jnp.float32),
        pltpu.VMEM((2, _K1), jnp.int32),
        pltpu.VMEM((2, _K1), jnp.int32),
        pltpu.VMEM((2, _K1, 16), jnp.float32),
        pltpu.VMEM((2, _K1), jnp.float32),
        pltpu.VMEM_SHARED((_NPAD1,), jnp.float32),
        [pltpu.SemaphoreType.DMA] * 2,
    ],
)
def _edge1(a_hbm, b_hbm, c_hbm, src_hbm, dst_hbm, bm_hbm, out_hbm,
           bm_v, a_sp, b_sp, zst, src_v, dst_v, bufc, bufm, agg_sh, sems):
    cc = lax.axis_index("c")
    s = lax.axis_index("s")
    w = cc * _NSUB + s
    base = w * _EPW

    pltpu.sync_copy(bm_hbm, bm_v)
    bmv = bm_v[pl.ds(0, 16)]
    pltpu.sync_copy(a_hbm, a_sp)
    pltpu.sync_copy(b_hbm, b_sp)

    def zbody(i, carry):
        zst[pl.ds(16 * i, 16)] = jnp.zeros((16,), jnp.float32)
        return carry
    lax.fori_loop(0, 40, zbody, 0)
    pltpu.sync_copy(zst, agg_sh.at[pl.ds(s * 640, 640)])
    plsc.subcore_barrier()

    def load(p, ci):
        off = base + ci * _K1
        pltpu.async_copy(src_hbm.at[pl.ds(off, _K1)], src_v.at[p], sems[p])
        pltpu.async_copy(dst_hbm.at[pl.ds(off, _K1)], dst_v.at[p], sems[p])
        pltpu.async_copy(c_hbm.at[pl.ds(off, _K1)], bufc.at[p], sems[p])

    def waitset(p):
        pltpu.make_async_copy(src_hbm.at[pl.ds(0, _K1)], src_v.at[p],
                              sems[p]).wait()
        pltpu.make_async_copy(src_hbm.at[pl.ds(0, _K1)], dst_v.at[p],
                              sems[p]).wait()
        pltpu.make_async_copy(c_hbm.at[pl.ds(0, _K1)], bufc.at[p],
                              sems[p]).wait()

    zeros16 = jnp.zeros((16,), jnp.int32)
    iota16 = lax.iota(jnp.int32, 16)

    def process(p):
        waitset(p)
        pfull = jnp.full((16,), p, jnp.int32)
        for g in range(_K1 // 16):
            sl = pl.ds(16 * g, 16)
            va = plsc.load_gather(a_sp, [src_v[p, sl]])
            vb = plsc.load_gather(b_sp, [dst_v[p, sl]])
            vc = plsc.load_gather(bufc, [pfull, iota16 + 16 * g, zeros16])
            m = ((va + vb) + vc) + bmv
            bufm[p, sl] = jnp.maximum(m, jnp.float32(0.0))
        pltpu.sync_copy(bufm.at[p], agg_sh.at[dst_v.at[p]], add=True)

    load(0, 0)
    load(1, 1)

    def body(i, carry):
        c0 = 2 * i
        process(0)
        load(0, c0 + 2)
        process(1)
        load(1, jnp.minimum(c0 + 3, _NCHUNK1 - 1))
        return carry
    lax.fori_loop(0, (_NCHUNK1 - 1) // 2, body, 0)
    process(0)
    waitset(1)

    plsc.subcore_barrier()
    pltpu.sync_copy(agg_sh.at[pl.ds(s * 640, 640)],
                    out_hbm.at[cc, pl.ds(s * 640, 640)])


def _ab_call(h, ws, wd, D):
    R = 2000

    def body(x_ref, ws_ref, wd_ref, a_ref, b_ref):
        xv = x_ref[...]
        a_ref[...] = jnp.dot(xv, ws_ref[...],
                             preferred_element_type=jnp.float32)
        b_ref[...] = jnp.dot(xv, wd_ref[...],
                             preferred_element_type=jnp.float32)

    return pl.pallas_call(
        body,
        grid=(_N // R,),
        in_specs=[
            pl.BlockSpec((R, 128), lambda i: (i, 0)),
            pl.BlockSpec((128, D), lambda i: (0, 0)),
            pl.BlockSpec((128, D), lambda i: (0, 0)),
        ],
        out_specs=[pl.BlockSpec((R, D), lambda i: (i, 0)),
                   pl.BlockSpec((R, D), lambda i: (i, 0))],
        out_shape=[jax.ShapeDtypeStruct((_N, D), jnp.float32)] * 2,
    )(h, ws, wd)


def _c_call(ea, we, D):
    RB = 4000

    def body(ea_ref, we_ref, c_ref):
        c_ref[...] = jnp.dot(ea_ref[...], we_ref[...],
                             preferred_element_type=jnp.float32)

    return pl.pallas_call(
        body,
        grid=(_E // RB,),
        in_specs=[pl.BlockSpec((RB, 16), lambda i: (i, 0)),
                  pl.BlockSpec((16, D), lambda i: (0, 0))],
        out_specs=pl.BlockSpec((RB, D), lambda i: (i, 0)),
        out_shape=jax.ShapeDtypeStruct((_E, D), jnp.float32),
    )(ea, we)


def _node_call(h, a0, a1, wn, bn, act, D):
    R = 2000

    def body(x_ref, a0_ref, a1_ref, wn_ref, bn_ref, o_ref):
        agg = a0_ref[...] + a1_ref[...]
        xcat = jnp.concatenate([x_ref[...], agg], axis=1)
        out = jnp.dot(xcat, wn_ref[...],
                      preferred_element_type=jnp.float32) + bn_ref[...]
        if act == "relu":
            out = jnp.maximum(out, jnp.float32(0.0))
        elif act == "sigmoid":
            out = jax.nn.sigmoid(out)
        o_ref[...] = out

    return pl.pallas_call(
        body,
        grid=(_N // R,),
        in_specs=[
            pl.BlockSpec((R, 128), lambda i: (i, 0)),
            pl.BlockSpec((R, D), lambda i: (i, 0)),
            pl.BlockSpec((R, D), lambda i: (i, 0)),
            pl.BlockSpec((128 + D, D), lambda i: (0, 0)),
            pl.BlockSpec((1, D), lambda i: (0, 0)),
        ],
        out_specs=pl.BlockSpec((R, D), lambda i: (i, 0)),
        out_shape=jax.ShapeDtypeStruct((_N, D), jnp.float32),
    )(h, a0, a1, wn, bn.reshape(1, D))


def kernel(x, edge_index, edge_attr, Wm1, bm1, Wn1, bn1, Wm2, bm2, Wn2, bn2,
           Wm3, bm3, Wn3, bn3, Wm4, bm4, Wn4, bn4):
    src = edge_index[0]
    dst = edge_index[1]

    h = x
    for Wm, bm, Wn, bn in ((Wm1, bm1, Wn1, bn1), (Wm2, bm2, Wn2, bn2),
                           (Wm3, bm3, Wn3, bn3)):
        ws, wd, we = Wm[:128], Wm[128:256], Wm[256:272]
        a, b = _ab_call(h, ws, wd, 128)
        cc = _c_call(edge_attr, we, 128)
        aggp = _edge128(a, b, cc, src, dst, bm)
        h = _node_call(h, aggp[0], aggp[1], Wn, bn, "relu", 128)

    ws4 = jnp.pad(Wm4[:128], ((0, 0), (0, 15)))
    wd4 = jnp.pad(Wm4[128:256], ((0, 0), (0, 15)))
    we4 = jnp.pad(Wm4[256:272], ((0, 0), (0, 15)))
    bm4b = jnp.tile(bm4, 16)
    wn4 = jnp.pad(Wn4, ((0, 15), (0, 15)))
    bn4p = jnp.pad(bn4, (0, 15))

    a, b = _ab_call(h, ws4, wd4, 16)
    cc = _c_call(edge_attr, we4, 16)
    aggp = _edge1(a[:, 0], b[:, 0], cc, src, dst, bm4b)
    a0 = jnp.pad(aggp[0, :_N, None], ((0, 0), (0, 15)))
    a1 = jnp.pad(aggp[1, :_N, None], ((0, 0), (0, 15)))
    out16 = _node_call(h, a0, a1, wn4, bn4p, "sigmoid", 16)
    return out16[:, 0:1]
